# speedup vs baseline: 1.3841x; 1.3841x over previous
import functools

import jax
import jax.numpy as jnp
from jax.experimental import pallas as pl
from jax.experimental.pallas import tpu as pltpu

EPS = 1e-6
TOK_BLK = 256


def _gelu_exact(y):
    return 0.5 * y * (1.0 + jax.lax.erf(y * 0.7071067811865476))


def _router_body(x_ref, wr_ref, c_ref, xn_ref, xnb_ref, aff_ref, mask_ref):
    xb = x_ref[...]
    ms = jnp.mean(xb * xb, axis=-1, keepdims=True)
    xn = wr_ref[...] * (xb * jax.lax.rsqrt(ms + EPS))
    xn_ref[...] = xn
    xnb_ref[...] = xn.astype(jnp.bfloat16)
    logits = jax.lax.dot_general(
        xn, c_ref[...], (((1,), (1,)), ((), ())),
        preferred_element_type=jnp.float32)
    m = jnp.max(logits, axis=-1, keepdims=True)
    ex = jnp.exp(logits - m)
    aff = ex / jnp.sum(ex, axis=-1, keepdims=True)
    aff_ref[...] = aff
    ne = aff.shape[-1]
    idx = jax.lax.broadcasted_iota(jnp.int32, aff.shape, 1)
    m1 = jnp.max(aff, axis=-1, keepdims=True)
    i1 = jnp.min(jnp.where(aff == m1, idx, ne), axis=-1, keepdims=True)
    oh1 = idx == i1
    a2 = jnp.where(oh1, -jnp.inf, aff)
    m2 = jnp.max(a2, axis=-1, keepdims=True)
    i2 = jnp.min(jnp.where(a2 == m2, idx, ne), axis=-1, keepdims=True)
    oh2 = idx == i2
    mask_ref[...] = jnp.where(oh1, m1, 0.0) + jnp.where(oh2, m2, 0.0)


def _routed_body(xn_ref, xnb_ref, mask_ref, w1_ref, w2_ref, br1_ref, br2_ref,
                 out_ref):
    n = pl.program_id(0)
    t = pl.program_id(1)
    blk = xnb_ref.shape[0]
    x = xnb_ref[...]
    h = jax.lax.dot_general(
        x, w1_ref[0], (((1,), (0,)), ((), ())),
        preferred_element_type=jnp.float32)
    h = h + br1_ref[pl.ds(n, 1), :]
    y = jax.lax.dot_general(
        h.astype(jnp.bfloat16), w2_ref[0], (((1,), (0,)), ((), ())),
        preferred_element_type=jnp.float32)
    y = y + br2_ref[pl.ds(n, 1), :]
    g = _gelu_exact(y)
    mask = mask_ref[...]
    lane = jax.lax.broadcasted_iota(jnp.int32, mask.shape, 1)
    w = jnp.sum(jnp.where(lane == n, mask, 0.0), axis=1, keepdims=True)
    rows = pl.ds(t * blk, blk)

    @pl.when(n == 0)
    def _():
        out_ref[rows, :] = xn_ref[...] + w * g

    @pl.when(n > 0)
    def _():
        out_ref[rows, :] += w * g


def _shared_body(xnb_ref, prev_ref, w1_ref, w2_ref, bs1_ref, bs2_ref, out_ref):
    x = xnb_ref[...]
    h = jax.lax.dot_general(
        x, w1_ref[0], (((1,), (0,)), ((), ())),
        preferred_element_type=jnp.float32)
    h = h + bs1_ref[...]
    y = jax.lax.dot_general(
        h.astype(jnp.bfloat16), w2_ref[0], (((1,), (0,)), ((), ())),
        preferred_element_type=jnp.float32)
    y = y + bs2_ref[...]
    out_ref[...] = prev_ref[...] + _gelu_exact(y)


def _impl(x, w_rms, Ws1, bs1, Ws2, bs2, Wr1, br1, Wr2, br2, centroids,
          interpret):
    b, s, d = x.shape
    nr, _, e = Wr1.shape
    xm = x.reshape(s, d)
    nt = s // TOK_BLK

    xn, xnb, aff, mask = pl.pallas_call(
        _router_body,
        grid=(nt,),
        in_specs=[
            pl.BlockSpec((TOK_BLK, d), lambda t: (t, 0)),
            pl.BlockSpec((1, d), lambda t: (0, 0)),
            pl.BlockSpec((nr, d), lambda t: (0, 0)),
        ],
        out_specs=[
            pl.BlockSpec((TOK_BLK, d), lambda t: (t, 0)),
            pl.BlockSpec((TOK_BLK, d), lambda t: (t, 0)),
            pl.BlockSpec((TOK_BLK, nr), lambda t: (t, 0)),
            pl.BlockSpec((TOK_BLK, nr), lambda t: (t, 0)),
        ],
        out_shape=[
            jax.ShapeDtypeStruct((s, d), jnp.float32),
            jax.ShapeDtypeStruct((s, d), jnp.bfloat16),
            jax.ShapeDtypeStruct((s, nr), jnp.float32),
            jax.ShapeDtypeStruct((s, nr), jnp.float32),
        ],
        interpret=interpret,
    )(xm, w_rms.reshape(1, d), centroids)

    w1b = Wr1.astype(jnp.bfloat16)
    w2b = Wr2.astype(jnp.bfloat16)
    ws1b = Ws1.astype(jnp.bfloat16)
    ws2b = Ws2.astype(jnp.bfloat16)

    out1 = pl.pallas_call(
        _routed_body,
        grid=(nr, nt),
        in_specs=[
            pl.BlockSpec((TOK_BLK, d), lambda n, t: (t, 0)),
            pl.BlockSpec((TOK_BLK, d), lambda n, t: (t, 0)),
            pl.BlockSpec((TOK_BLK, nr), lambda n, t: (t, 0)),
            pl.BlockSpec((1, d, e), lambda n, t: (n, 0, 0)),
            pl.BlockSpec((1, e, d), lambda n, t: (n, 0, 0)),
            pl.BlockSpec((nr, e), lambda n, t: (0, 0)),
            pl.BlockSpec((nr, d), lambda n, t: (0, 0)),
        ],
        out_specs=pl.BlockSpec((s, d), lambda n, t: (0, 0)),
        out_shape=jax.ShapeDtypeStruct((s, d), jnp.float32),
        interpret=interpret,
    )(xn, xnb, mask, w1b, w2b, br1, br2)

    out = pl.pallas_call(
        _shared_body,
        grid=(nt,),
        in_specs=[
            pl.BlockSpec((TOK_BLK, d), lambda t: (t, 0)),
            pl.BlockSpec((TOK_BLK, d), lambda t: (t, 0)),
            pl.BlockSpec((1, d, e), lambda t: (0, 0, 0)),
            pl.BlockSpec((1, e, d), lambda t: (0, 0, 0)),
            pl.BlockSpec((1, e), lambda t: (0, 0)),
            pl.BlockSpec((1, d), lambda t: (0, 0)),
        ],
        out_specs=pl.BlockSpec((TOK_BLK, d), lambda t: (t, 0)),
        out_shape=jax.ShapeDtypeStruct((s, d), jnp.float32),
        interpret=interpret,
    )(xnb, out1, ws1b, ws2b, bs1, bs2)

    return out.reshape(b, s, d), aff


def kernel(x, w_rms, Ws1, bs1, Ws2, bs2, Wr1, br1, Wr2, br2, centroids):
    return _impl(x, w_rms, Ws1, bs1, Ws2, bs2, Wr1, br1, Wr2, br2, centroids,
                 interpret=False)
